# baseline (speedup 1.0000x reference)
import jax
import jax.numpy as jnp
from jax.experimental import pallas as pl
from jax.experimental.pallas import tpu as pltpu

_N = 5000
_K = 12544
_H = 1024
_BN = 512
_BK = 1792
_NK = _K // _BK
_NN = 10
_J = 7
_BJ = _BK // _J

_DN = (((1,), (1,)), ((), ()))


def _body(*refs):
    x_refs = refs[:_J]
    w1_refs = refs[_J:2 * _J]
    (b1_ref, w2_ref, b2_ref, wc_ref, bc_ref, wr_ref, br_ref,
     cls_ref, box_ref, acc_ref, w1b_ref) = refs[2 * _J:]
    k = pl.program_id(0)
    n = pl.program_id(1)

    @pl.when(n == 0)
    def _pack_w1():
        w1b_ref[...] = jnp.concatenate(
            [r[...].astype(jnp.bfloat16) for r in w1_refs], axis=1)

    xb = jnp.concatenate(
        [r[...].astype(jnp.bfloat16) for r in x_refs], axis=1)
    part = jax.lax.dot_general(
        xb, w1b_ref[...], _DN, preferred_element_type=jnp.float32)

    @pl.when(k == 0)
    def _init():
        acc_ref[n] = part

    @pl.when(k > 0)
    def _accum():
        acc_ref[n] += part

    @pl.when(k == _NK - 1)
    def _finish():
        h1 = jnp.maximum(acc_ref[n] + b1_ref[...], 0.0).astype(jnp.bfloat16)
        h2 = jax.lax.dot_general(
            h1, w2_ref[...], _DN, preferred_element_type=jnp.float32)
        h2 = jnp.maximum(h2 + b2_ref[...], 0.0).astype(jnp.bfloat16)
        cls_ref[...] = jax.lax.dot_general(
            h2, wc_ref[...], _DN,
            preferred_element_type=jnp.float32) + bc_ref[...]
        box_ref[...] = jax.lax.dot_general(
            h2, wr_ref[...], _DN,
            preferred_element_type=jnp.float32) + br_ref[...]


def _x_spec(j):
    return pl.BlockSpec((_BN, _BJ), lambda k, n, j=j: (n, k * _J + j))


def _w1_spec(j):
    return pl.BlockSpec((_H, _BJ), lambda k, n, j=j: (0, k * _J + j))


def kernel(feature_vectors, W1, b1, W2, b2, Wc, bc, Wr, br):
    c1 = Wc.shape[0]
    c4 = Wr.shape[0]
    cls_out, box_out = pl.pallas_call(
        _body,
        grid=(_NK, _NN),
        in_specs=(
            [_x_spec(j) for j in range(_J)]
            + [_w1_spec(j) for j in range(_J)]
            + [
                pl.BlockSpec((1, _H), lambda k, n: (0, 0)),
                pl.BlockSpec((_H, _H), lambda k, n: (0, 0)),
                pl.BlockSpec((1, _H), lambda k, n: (0, 0)),
                pl.BlockSpec((c1, _H), lambda k, n: (0, 0)),
                pl.BlockSpec((1, c1), lambda k, n: (0, 0)),
                pl.BlockSpec((c4, _H), lambda k, n: (0, 0)),
                pl.BlockSpec((1, c4), lambda k, n: (0, 0)),
            ]
        ),
        out_specs=[
            pl.BlockSpec((_BN, c1), lambda k, n: (n, 0)),
            pl.BlockSpec((_BN, c4), lambda k, n: (n, 0)),
        ],
        out_shape=[
            jax.ShapeDtypeStruct((_N, c1), jnp.float32),
            jax.ShapeDtypeStruct((_N, c4), jnp.float32),
        ],
        scratch_shapes=[
            pltpu.VMEM((_NN, _BN, _H), jnp.float32),
            pltpu.VMEM((_H, _BK), jnp.bfloat16),
        ],
        compiler_params=pltpu.CompilerParams(
            dimension_semantics=("arbitrary", "arbitrary")),
    )(*([feature_vectors] * _J), *([W1] * _J), b1.reshape(1, -1),
      W2.astype(jnp.bfloat16), b2.reshape(1, -1), Wc.astype(jnp.bfloat16),
      bc.reshape(1, -1), Wr.astype(jnp.bfloat16), br.reshape(1, -1))
    return (cls_out, box_out)


# speedup vs baseline: 1.2274x; 1.2274x over previous
import jax
import jax.numpy as jnp
from jax.experimental import pallas as pl
from jax.experimental.pallas import tpu as pltpu

_N = 5000
_K = 12544
_H = 1024
_BN = 1280
_BK = 1792
_NK = _K // _BK
_NN = 4

_DN = (((1,), (1,)), ((), ()))


def _body(x_ref, w1_ref, b1_ref, w2_ref, b2_ref, wc_ref, bc_ref, wr_ref,
          br_ref, cls_ref, box_ref, acc_ref):
    k = pl.program_id(1)

    part = jax.lax.dot_general(
        x_ref[...], w1_ref[...], _DN, preferred_element_type=jnp.float32)

    @pl.when(k == 0)
    def _init():
        acc_ref[...] = part

    @pl.when(k > 0)
    def _accum():
        acc_ref[...] += part

    @pl.when(k == _NK - 1)
    def _finish():
        h1 = jnp.maximum(acc_ref[...] + b1_ref[...], 0.0)
        h2 = jax.lax.dot_general(
            h1, w2_ref[...], _DN, preferred_element_type=jnp.float32)
        h2 = jnp.maximum(h2 + b2_ref[...], 0.0)
        cls_ref[...] = jax.lax.dot_general(
            h2, wc_ref[...], _DN,
            preferred_element_type=jnp.float32) + bc_ref[...]
        box_ref[...] = jax.lax.dot_general(
            h2, wr_ref[...], _DN,
            preferred_element_type=jnp.float32) + br_ref[...]


def kernel(feature_vectors, W1, b1, W2, b2, Wc, bc, Wr, br):
    c1 = Wc.shape[0]
    c4 = Wr.shape[0]
    cls_out, box_out = pl.pallas_call(
        _body,
        grid=(_NN, _NK),
        in_specs=[
            pl.BlockSpec((_BN, _BK), lambda n, k: (n, k)),
            pl.BlockSpec((_H, _BK), lambda n, k: (0, k)),
            pl.BlockSpec((1, _H), lambda n, k: (0, 0)),
            pl.BlockSpec((_H, _H), lambda n, k: (0, 0)),
            pl.BlockSpec((1, _H), lambda n, k: (0, 0)),
            pl.BlockSpec((c1, _H), lambda n, k: (0, 0)),
            pl.BlockSpec((1, c1), lambda n, k: (0, 0)),
            pl.BlockSpec((c4, _H), lambda n, k: (0, 0)),
            pl.BlockSpec((1, c4), lambda n, k: (0, 0)),
        ],
        out_specs=[
            pl.BlockSpec((_BN, c1), lambda n, k: (n, 0)),
            pl.BlockSpec((_BN, c4), lambda n, k: (n, 0)),
        ],
        out_shape=[
            jax.ShapeDtypeStruct((_N, c1), jnp.float32),
            jax.ShapeDtypeStruct((_N, c4), jnp.float32),
        ],
        scratch_shapes=[pltpu.VMEM((_BN, _H), jnp.float32)],
        compiler_params=pltpu.CompilerParams(
            dimension_semantics=("parallel", "arbitrary")),
    )(feature_vectors, W1, b1.reshape(1, -1), W2.astype(jnp.bfloat16),
      b2.reshape(1, -1), Wc.astype(jnp.bfloat16), bc.reshape(1, -1),
      Wr.astype(jnp.bfloat16), br.reshape(1, -1))
    return (cls_out, box_out)
